# Initial kernel scaffold; baseline (speedup 1.0000x reference)
import jax
import jax.numpy as jnp
from jax.experimental import pallas as pl


def kernel(cleavage_indices, pos_embed):
    raise NotImplementedError("write your pallas kernel here")


# speedup vs baseline: 3.2460x; 3.2460x over previous
import functools

import jax
import jax.numpy as jnp
from jax import lax
from jax.experimental import pallas as pl
from jax.experimental.pallas import tpu as pltpu
from jax.experimental.pallas import tpu_sc as plsc

EMBED = 128
CHUNK = 128


@functools.partial(jax.jit, static_argnums=(2, 3))
def _sc_gather(table, idx3, nw, k_per_w):
    mesh = plsc.VectorSubcoreMesh(core_axis_name="c", subcore_axis_name="s")
    total = nw * k_per_w * CHUNK

    @functools.partial(
        pl.kernel,
        mesh=mesh,
        out_type=jax.ShapeDtypeStruct((total, EMBED), jnp.float32),
        scratch_types=[
            pltpu.VMEM((k_per_w, CHUNK), jnp.int32),
            pltpu.VMEM((2, CHUNK, EMBED), jnp.float32),
            pltpu.SemaphoreType.DMA,
            pltpu.SemaphoreType.DMA,
        ],
    )
    def k(table_hbm, idx_hbm, out_hbm, idx_v, rows_v, gsem, ssem):
        nc = 2
        wid = lax.axis_index("s") * nc + lax.axis_index("c")
        base = wid * (k_per_w * CHUNK)
        pltpu.sync_copy(idx_hbm.at[wid], idx_v)

        pltpu.async_copy(table_hbm.at[idx_v.at[0]], rows_v.at[0], gsem)

        def body(j, _):
            slot = lax.rem(j, 2)
            nxt = lax.rem(j + 1, 2)

            @pl.when(j + 1 < k_per_w)
            def _():
                pltpu.async_copy(table_hbm.at[idx_v.at[j + 1]], rows_v.at[nxt], gsem)

            pltpu.make_async_copy(table_hbm.at[idx_v.at[j]], rows_v.at[slot], gsem).wait()
            out_slice = out_hbm.at[pl.ds(base + j * CHUNK, CHUNK)]
            pltpu.async_copy(rows_v.at[slot], out_slice, ssem)
            pltpu.make_async_copy(rows_v.at[slot], out_slice, ssem).wait()
            return 0

        lax.fori_loop(0, k_per_w, body, 0)

    return k(table, idx3)


def kernel(cleavage_indices, pos_embed):
    b, s = cleavage_indices.shape
    total = b * s
    info = plsc.get_sparse_core_info()
    nw = info.num_cores * info.num_subcores
    k_per_w = total // (nw * CHUNK)
    idx3 = cleavage_indices.astype(jnp.int32).reshape(nw, k_per_w, CHUNK)
    out = _sc_gather(pos_embed, idx3, nw, k_per_w)
    return out.reshape(b, s, EMBED)


# speedup vs baseline: 3.2493x; 1.0010x over previous
import functools

import jax
import jax.numpy as jnp
from jax import lax
from jax.experimental import pallas as pl
from jax.experimental.pallas import tpu as pltpu
from jax.experimental.pallas import tpu_sc as plsc

EMBED = 128
CHUNK = 128
NB = 5
GD = 2
SD = 3


@functools.partial(jax.jit, static_argnums=(2, 3))
def _sc_gather(table, idx3, nw, k_per_w):
    mesh = plsc.VectorSubcoreMesh(core_axis_name="c", subcore_axis_name="s")
    total = nw * k_per_w * CHUNK
    assert k_per_w % NB == 0 and k_per_w >= NB

    @functools.partial(
        pl.kernel,
        mesh=mesh,
        out_type=jax.ShapeDtypeStruct((total, EMBED), jnp.float32),
        scratch_types=[
            pltpu.VMEM((k_per_w, CHUNK), jnp.int32),
            pltpu.VMEM((NB, CHUNK, EMBED), jnp.float32),
            pltpu.SemaphoreType.DMA((NB,)),
            pltpu.SemaphoreType.DMA((NB,)),
        ],
    )
    def k(table_hbm, idx_hbm, out_hbm, idx_v, rows_v, gsem, ssem):
        nc = 2
        wid = lax.axis_index("s") * nc + lax.axis_index("c")
        base = wid * (k_per_w * CHUNK)
        pltpu.sync_copy(idx_hbm.at[wid], idx_v)

        def gather(j, b):
            return pltpu.make_async_copy(
                table_hbm.at[idx_v.at[j]], rows_v.at[b], gsem.at[b])

        def store(j, b):
            return pltpu.make_async_copy(
                rows_v.at[b], out_hbm.at[pl.ds(base + j * CHUNK, CHUNK)], ssem.at[b])

        for b in range(GD):
            gather(b, b).start()

        def outer(i, _):
            g = i * NB
            for b in range(NB):
                j = g + b
                bg = (b + GD) % NB
                @pl.when(j + GD - NB >= 0)
                def _():
                    store(j + GD - NB, bg).wait()

                @pl.when(j + GD < k_per_w)
                def _():
                    gather(j + GD, bg).start()

                gather(j, b).wait()
                store(j, b).start()
            return 0

        lax.fori_loop(0, k_per_w // NB, outer, 0)

        for j in range(k_per_w - SD, k_per_w):
            store(j, j % NB).wait()

    return k(table, idx3)


def kernel(cleavage_indices, pos_embed):
    b, s = cleavage_indices.shape
    total = b * s
    info = plsc.get_sparse_core_info()
    nw = info.num_cores * info.num_subcores
    k_per_w = total // (nw * CHUNK)
    idx3 = cleavage_indices.astype(jnp.int32).reshape(nw, k_per_w, CHUNK)
    out = _sc_gather(pos_embed, idx3, nw, k_per_w)
    return out.reshape(b, s, EMBED)
